# Initial kernel scaffold; baseline (speedup 1.0000x reference)
import jax
import jax.numpy as jnp
from jax.experimental import pallas as pl


def kernel(x, table, a0, b0, a1, b1):
    raise NotImplementedError("write your pallas kernel here")


# speedup vs baseline: 1.2927x; 1.2927x over previous
import jax
import jax.numpy as jnp
from jax import lax
from jax.experimental import pallas as pl
from jax.experimental.pallas import tpu as pltpu
from jax.experimental.pallas import tpu_sc as plsc

ROWS = 1000000
DIM = 64
N_CHUNKS = 4
BATCH = 16384
PRIME = (1 << 31) - 1

NC = 2
NS = 16
LANES = 16
NW = NC * NS
B_PER_W = BATCH // NW
PIECE = 64
N_PIECE = B_PER_W // PIECE
SCALE = (N_CHUNKS * DIM) ** 0.5 / N_CHUNKS

_U = jnp.uint32


def _modp(r):
    return jnp.minimum(r, r - _U(PRIME))


def _shift_modp(v, k):
    lo = (v << _U(k)) & _U(PRIME)
    hi = v >> _U(31 - k)
    return _modp(lo + hi)


def _poly_hash_vec(xv, a2, a1, a0, bv):
    t2 = _shift_modp(_modp(a2 * xv), 22)
    t1 = _shift_modp(_modp(a1 * xv), 11)
    t0 = _modp(a0 * xv)
    s = _modp(t2 + t1)
    s = _modp(s + t0)
    return _modp(s + bv)


def _sc_body(x_hbm, table_hbm, params_hbm, out_hbm,
             x_v, params_v, h0_v, h1r_v, h1c_v, w_v, rows_v, wrows_v,
             out_v, sem_r0, sem_r1, sem_o0, sem_o1):
    wid = lax.axis_index("s") * NC + lax.axis_index("c")
    base = wid * B_PER_W

    pltpu.sync_copy(x_hbm.at[pl.ds(base, B_PER_W)], x_v)
    pltpu.sync_copy(params_hbm, params_v)

    coef = [[[params_v[f, c, j] for j in range(4)]
             for c in range(N_CHUNKS)] for f in range(2)]

    def hash_piece(p):
        def body(j, _):
            xv = x_v[pl.ds(p * PIECE + j * LANES, LANES)].astype(_U)
            for c in range(N_CHUNKS):
                c0 = coef[0][c]
                h0 = _poly_hash_vec(xv, c0[0].astype(_U), c0[1].astype(_U),
                                    c0[2].astype(_U), c0[3].astype(_U))
                h0 = h0 % _U(ROWS)
                c1 = coef[1][c]
                h1 = _poly_hash_vec(xv, c1[0].astype(_U), c1[1].astype(_U),
                                    c1[2].astype(_U), c1[3].astype(_U))
                h1 = h1 % _U(ROWS * DIM)
                sl = pl.ds(j * LANES, LANES)
                h0_v[c, p, sl] = h0.astype(jnp.int32)
                h1r_v[c, p, sl] = (h1 >> _U(6)).astype(jnp.int32)
                h1c_v[c, p, sl] = (h1 & _U(DIM - 1)).astype(jnp.int32)
            return _

        lax.fori_loop(jnp.int32(0), jnp.int32(PIECE // LANES), body,
                      jnp.int32(0))

    for p in range(N_PIECE):
        hash_piece(p)

    sem_r = [sem_r0, sem_r1]
    sem_o = [sem_o0, sem_o1]

    def fire(p):
        buf = p % 2
        cps = []
        for c in range(N_CHUNKS):
            cps.append(pltpu.async_copy(table_hbm.at[h0_v.at[c, p]],
                                        rows_v.at[buf, c], sem_r[buf]))
            cps.append(pltpu.async_copy(table_hbm.at[h1r_v.at[c, p]],
                                        wrows_v.at[buf, c], sem_r[buf]))
        return cps

    cps = fire(0)
    out_cps = [None, None]
    lane = lax.iota(jnp.int32, LANES)
    for p in range(N_PIECE):
        buf = p % 2
        for cp in cps:
            cp.wait()
        if p + 1 < N_PIECE:
            cps = fire(p + 1)
        if out_cps[buf] is not None:
            out_cps[buf].wait()
            out_cps[buf] = None

        bufv = jnp.full((LANES,), buf, jnp.int32)
        for c in range(N_CHUNKS):
            cv = jnp.full((LANES,), c, jnp.int32)
            for g in range(PIECE // LANES):
                cols = h1c_v[c, p, pl.ds(g * LANES, LANES)]
                wv = plsc.load_gather(
                    wrows_v, [bufv, cv, lane + jnp.int32(g * LANES), cols])
                w_v[c, pl.ds(p * PIECE + g * LANES, LANES)] = wv * SCALE

        def body(b, _):
            bidx = jnp.full((LANES,), p * PIECE + b, jnp.int32)
            wv = [plsc.load_gather(
                      w_v, [jnp.full((LANES,), c, jnp.int32), bidx])
                  for c in range(N_CHUNKS)]
            for d in range(DIM // LANES):
                sl = pl.ds(d * LANES, LANES)
                acc = wv[0] * rows_v[buf, 0, b, sl]
                for c in range(1, N_CHUNKS):
                    acc = acc + wv[c] * rows_v[buf, c, b, sl]
                out_v[buf, b, sl] = acc
            return _

        lax.fori_loop(jnp.int32(0), jnp.int32(PIECE), body, jnp.int32(0))
        out_cps[buf] = pltpu.async_copy(
            out_v.at[buf], out_hbm.at[pl.ds(base + p * PIECE, PIECE)],
            sem_o[buf])

    for cp in out_cps:
        if cp is not None:
            cp.wait()


@jax.jit
def _wh_embed(x_i32, table, params):
    mesh = plsc.VectorSubcoreMesh(core_axis_name="c", subcore_axis_name="s")
    f = pl.kernel(
        _sc_body,
        out_type=jax.ShapeDtypeStruct((BATCH, DIM), jnp.float32),
        mesh=mesh,
        scratch_types=[
            pltpu.VMEM((B_PER_W,), jnp.int32),
            pltpu.VMEM((2, N_CHUNKS, 4, LANES), jnp.int32),
            pltpu.VMEM((N_CHUNKS, N_PIECE, PIECE), jnp.int32),
            pltpu.VMEM((N_CHUNKS, N_PIECE, PIECE), jnp.int32),
            pltpu.VMEM((N_CHUNKS, N_PIECE, PIECE), jnp.int32),
            pltpu.VMEM((N_CHUNKS, B_PER_W), jnp.float32),
            pltpu.VMEM((2, N_CHUNKS, PIECE, DIM), jnp.float32),
            pltpu.VMEM((2, N_CHUNKS, PIECE, DIM), jnp.float32),
            pltpu.VMEM((2, PIECE, DIM), jnp.float32),
            pltpu.SemaphoreType.DMA,
            pltpu.SemaphoreType.DMA,
            pltpu.SemaphoreType.DMA,
            pltpu.SemaphoreType.DMA,
        ],
        compiler_params=pltpu.CompilerParams(needs_layout_passes=False,
                                             use_tc_tiling_on_sc=False),
    )
    return f(x_i32, table, params)


def kernel(x, table, a0, b0, a1, b1):
    x_i32 = x.astype(jnp.int32)
    p = []
    for a, b in ((a0, b0), (a1, b1)):
        a = a.astype(jnp.int64)
        p.append(jnp.stack([a >> 22, (a >> 11) & 2047, a & 2047,
                            b.astype(jnp.int64)], axis=-1))
    params = jnp.stack(p).astype(jnp.int32)
    params = jnp.broadcast_to(params[..., None],
                              (2, N_CHUNKS, 4, LANES))
    with jax.enable_x64(False):
        return _wh_embed(x_i32, table, params)
